# Initial kernel scaffold; baseline (speedup 1.0000x reference)
import jax
import jax.numpy as jnp
from jax import lax
from jax.experimental import pallas as pl
from jax.experimental.pallas import tpu as pltpu
from jax.experimental.pallas import tpu_sc as plsc

BATCH = 1024
SEQ = 200
D = 128
NC = 2
NS = 16
NW = NC * NS
SEQ_PER_W = BATCH // NW
HALF = SEQ // 2


def _body(x_hbm, tok_hbm, pos_hbm, out_hbm, idx_v, rows_v, pos_v, gsem):
    wid = lax.axis_index("s") * NC + lax.axis_index("c")
    base = wid * SEQ_PER_W
    pltpu.sync_copy(x_hbm.at[pl.ds(base, SEQ_PER_W)], idx_v)
    pltpu.sync_copy(pos_hbm, pos_v)

    def seq_body(i, carry):
        pltpu.sync_copy(pos_v, rows_v)
        pltpu.async_copy(
            tok_hbm.at[idx_v.at[i, pl.ds(0, HALF)]],
            rows_v.at[pl.ds(0, HALF)], gsem, add=True).wait()
        pltpu.async_copy(
            tok_hbm.at[idx_v.at[i, pl.ds(HALF, HALF)]],
            rows_v.at[pl.ds(HALF, HALF)], gsem, add=True).wait()
        pltpu.sync_copy(rows_v, out_hbm.at[base + i])
        return carry

    lax.fori_loop(0, SEQ_PER_W, seq_body, 0)


def kernel(x, token_table, pos_table):
    mesh = plsc.VectorSubcoreMesh(core_axis_name="c", subcore_axis_name="s")
    f = pl.kernel(
        _body,
        out_type=jax.ShapeDtypeStruct((BATCH, SEQ, D), jnp.float32),
        mesh=mesh,
        scratch_types=[
            pltpu.VMEM((SEQ_PER_W, SEQ), jnp.int32),
            pltpu.VMEM((SEQ, D), jnp.float32),
            pltpu.VMEM((SEQ, D), jnp.float32),
            pltpu.SemaphoreType.DMA,
        ],
    )
    return f(x, token_table, pos_table)


# speedup vs baseline: 2.8787x; 2.8787x over previous
import jax
import jax.numpy as jnp
from jax import lax
from jax.experimental import pallas as pl
from jax.experimental.pallas import tpu as pltpu
from jax.experimental.pallas import tpu_sc as plsc

BATCH = 1024
SEQ = 200
D = 128
NC = 2
NS = 16
NW = NC * NS
SEQ_PER_W = BATCH // NW
IDX_PER_W = SEQ_PER_W * SEQ
C1 = 104
C2 = SEQ - C1


def _body(x_hbm, tok_hbm, pos_hbm, out_hbm, idx_v, rows_v, gsem):
    wid = lax.axis_index("s") * NC + lax.axis_index("c")
    base = wid * SEQ_PER_W
    pltpu.sync_copy(
        x_hbm.at[pl.ds(pl.multiple_of(wid * IDX_PER_W, 8), IDX_PER_W)], idx_v)

    def seq_body(i, carry):
        off = pl.multiple_of(i * SEQ, 8)
        pltpu.sync_copy(pos_hbm, rows_v)
        pltpu.async_copy(
            tok_hbm.at[idx_v.at[pl.ds(off, C1)]],
            rows_v.at[pl.ds(0, C1)], gsem, add=True).wait()
        pltpu.async_copy(
            tok_hbm.at[idx_v.at[pl.ds(pl.multiple_of(off + C1, 8), C2)]],
            rows_v.at[pl.ds(C1, C2)], gsem, add=True).wait()
        pltpu.sync_copy(rows_v, out_hbm.at[base + i])
        return carry

    lax.fori_loop(0, SEQ_PER_W, seq_body, 0)


def kernel(x, token_table, pos_table):
    mesh = plsc.VectorSubcoreMesh(core_axis_name="c", subcore_axis_name="s")
    f = pl.kernel(
        _body,
        out_type=jax.ShapeDtypeStruct((BATCH, SEQ, D), jnp.float32),
        mesh=mesh,
        scratch_types=[
            pltpu.VMEM((IDX_PER_W,), jnp.int32),
            pltpu.VMEM((SEQ, D), jnp.float32),
            pltpu.SemaphoreType.DMA,
        ],
    )
    return f(x.reshape(-1), token_table, pos_table)


# speedup vs baseline: 7.3989x; 2.5702x over previous
import jax
import jax.numpy as jnp
from jax import lax
from jax.experimental import pallas as pl
from jax.experimental.pallas import tpu as pltpu
from jax.experimental.pallas import tpu_sc as plsc

BATCH = 1024
SEQ = 200
D = 128
NC = 2
NS = 16
NW = NC * NS
SEQ_PER_W = BATCH // NW
IDX_PER_W = SEQ_PER_W * SEQ
SPB = 2
ROWS = SPB * SEQ
NBLK = SEQ_PER_W // SPB
CHUNKS = []
for _s in range(SPB):
    CHUNKS += [(_s * SEQ, 104), (_s * SEQ + 104, 96)]


def _body(x_hbm, tok_hbm, pos_hbm, out_hbm, idx_v, rows, psh, sems):
    sid = lax.axis_index("s")
    wid = sid * NC + lax.axis_index("c")
    ibase = pl.multiple_of(wid * IDX_PER_W, 8)
    pltpu.sync_copy(x_hbm.at[pl.ds(ibase, IDX_PER_W)], idx_v)
    @pl.when(sid == 0)
    def _stage():
        pltpu.sync_copy(pos_hbm, rows[0].at[pl.ds(0, SEQ)])
        pltpu.sync_copy(rows[0].at[pl.ds(0, SEQ)], psh.at[pl.ds(0, SEQ)])
        pltpu.sync_copy(rows[0].at[pl.ds(0, SEQ)], psh.at[pl.ds(SEQ, SEQ)])
    plsc.subcore_barrier()

    psem, gsem, ssem = sems

    def launch(i):
        b = i % 2
        pltpu.async_copy(psh, rows[b], psem[b]).wait()
        for off, ln in CHUNKS:
            pltpu.async_copy(
                tok_hbm.at[idx_v.at[pl.ds(i * ROWS + off, ln)]],
                rows[b].at[pl.ds(off, ln)], gsem[b], add=True)

    def finish(i):
        b = i % 2
        for off, ln in CHUNKS:
            pltpu.make_async_copy(
                tok_hbm.at[idx_v.at[pl.ds(i * ROWS + off, ln)]],
                rows[b].at[pl.ds(off, ln)], gsem[b]).wait()
        pltpu.async_copy(
            rows[b],
            out_hbm.at[pl.ds(pl.multiple_of(wid * IDX_PER_W + i * ROWS, 8),
                             ROWS)],
            ssem[b])

    launch(0)
    for i in range(NBLK):
        if i + 1 < NBLK:
            if i + 1 >= 2:
                pltpu.make_async_copy(
                    rows[(i + 1) % 2],
                    out_hbm.at[pl.ds(0, ROWS)],
                    ssem[(i + 1) % 2]).wait()
            launch(i + 1)
        finish(i)
    for b in (NBLK % 2, (NBLK + 1) % 2):
        pltpu.make_async_copy(
            rows[b], out_hbm.at[pl.ds(0, ROWS)], ssem[b]).wait()


def kernel(x, token_table, pos_table):
    mesh = plsc.VectorSubcoreMesh(core_axis_name="c", subcore_axis_name="s")
    f = pl.kernel(
        _body,
        out_type=jax.ShapeDtypeStruct((BATCH * SEQ, D), jnp.float32),
        mesh=mesh,
        scratch_types=[
            pltpu.VMEM((IDX_PER_W,), jnp.int32),
            [pltpu.VMEM((ROWS, D), jnp.float32) for _ in range(2)],
            pltpu.VMEM_SHARED((ROWS, D), jnp.float32),
            [[pltpu.SemaphoreType.DMA for _ in range(2)] for _ in range(3)],
        ],
    )
    out = f(x.reshape(-1), token_table, pos_table)
    return out.reshape(BATCH, SEQ, D)
